# baseline (speedup 1.0000x reference)
import functools

import jax
import jax.numpy as jnp
from jax import lax
from jax.experimental import pallas as pl
from jax.experimental.pallas import tpu as pltpu
from jax.experimental.pallas import tpu_sc as plsc

_BATCH = 16384
_HIST = 200
_EMBED = 32
_B = _BATCH * _HIST
_LANES = 128
_SUB = 8
_CHUNK = _SUB * _LANES
_NW = 32
_NCHUNKS = _B // _CHUNK
_CPW = _NCHUNKS // _NW
_HG = _HIST // _SUB
_EG = _EMBED // 8


def _body(idx_hbm, table_hbm, out_hbm, idx_v, rows_v, tp_v,
          sem_i0, sem_i1, sem_g0, sem_g1, sem_t0, sem_t1):
    nc = plsc.get_sparse_core_info().num_cores
    wid = lax.axis_index("s") * nc + lax.axis_index("c")
    sem_i = (sem_i0, sem_i1)
    sem_g = (sem_g0, sem_g1)
    sem_t = (sem_t0, sem_t1)
    iota = lax.iota(jnp.int32, 16)
    bg_idx = [iota + bg * 16 for bg in range(_SUB)]

    def chunk_id(ch):
        return wid * _CPW + ch

    def start_idx(ch, slot):
        pltpu.async_copy(
            idx_hbm.at[pl.ds(chunk_id(ch) * _SUB, _SUB)], idx_v.at[slot],
            sem_i[slot])

    def wait_idx(slot):
        pltpu.make_async_copy(
            idx_hbm.at[pl.ds(0, _SUB)], idx_v.at[slot], sem_i[slot]).wait()

    def fire(slot):
        for j in range(_SUB):
            pltpu.async_copy(
                table_hbm.at[idx_v.at[slot, j]],
                rows_v.at[slot, pl.ds(j * _LANES, _LANES)],
                sem_g[slot])

    def drain_gathers(slot):
        pltpu.make_async_copy(
            table_hbm.at[pl.ds(0, _CHUNK)], rows_v.at[slot],
            sem_g[slot]).wait()

    def wait_tp(p):
        for _eg in range(_EG):
            pltpu.make_async_copy(
                tp_v.at[p, 0], out_hbm.at[0, 0, 0], sem_t[p]).wait()

    def transpose_store(ch, slot, g):
        cid = chunk_id(ch)
        hg = cid // _LANES
        bt = cid - hg * _LANES
        for h8 in range(_SUB):
            p = h8 % 2
            if h8 >= 2:
                wait_tp(p)
            else:
                @pl.when(g >= 1)
                def _():
                    wait_tp(p)
            block = rows_v.at[slot, pl.ds(h8 * _LANES, _LANES)]

            def tbody(e, carry):
                fe = jnp.full((16,), 0, jnp.int32) + e
                eg = e // 8
                e8 = e - eg * 8
                for bg in range(_SUB):
                    v = plsc.load_gather(block, [bg_idx[bg], fe])
                    tp_v[p, eg, e8, pl.ds(bg * 16, 16)] = v
                return carry

            lax.fori_loop(0, _EMBED, tbody, 0)
            h = hg * _SUB + h8
            for eg in range(_EG):
                pltpu.async_copy(
                    tp_v.at[p, eg], out_hbm.at[h, eg, bt], sem_t[p])

    def step(g, slot):
        @pl.when(g + 1 < _CPW)
        def _():
            wait_idx(slot ^ 1)
            fire(slot ^ 1)

        drain_gathers(slot)

        @pl.when(g + 2 < _CPW)
        def _():
            start_idx(g + 2, slot)

        transpose_store(g, slot, g)

    start_idx(0, 0)
    start_idx(1, 1)
    wait_idx(0)
    fire(0)

    def loop_body(i, carry):
        step(2 * i, 0)
        step(2 * i + 1, 1)
        return carry

    lax.fori_loop(0, _CPW // 2, loop_body, 0)
    wait_tp(0)
    wait_tp(1)


@functools.partial(jax.jit, static_argnames=())
def kernel(action, table):
    act_view = (action.astype(jnp.int32)
                .reshape(_LANES, _LANES, _HG, _SUB)
                .transpose(2, 0, 3, 1)
                .reshape(_B // _LANES, _LANES))
    mesh = plsc.VectorSubcoreMesh(core_axis_name="c", subcore_axis_name="s")
    out4 = pl.kernel(
        _body,
        out_type=jax.ShapeDtypeStruct((_HIST, _EG, _LANES, 8, _LANES),
                                      jnp.float32),
        mesh=mesh,
        scratch_types=[
            pltpu.VMEM((2, _SUB, _LANES), jnp.int32),
            pltpu.VMEM((2, _CHUNK, _EMBED), jnp.float32),
            pltpu.VMEM((2, _EG, 8, _LANES), jnp.float32),
            pltpu.SemaphoreType.DMA,
            pltpu.SemaphoreType.DMA,
            pltpu.SemaphoreType.DMA,
            pltpu.SemaphoreType.DMA,
            pltpu.SemaphoreType.DMA,
            pltpu.SemaphoreType.DMA,
        ],
        compiler_params=pltpu.CompilerParams(use_tc_tiling_on_sc=False,
                                             needs_layout_passes=False),
    )(act_view, table)
    return (out4.transpose(2, 4, 0, 1, 3)
            .reshape(_BATCH, _HIST, _EMBED))


# speedup vs baseline: 1.1894x; 1.1894x over previous
import functools

import jax
import jax.numpy as jnp
from jax import lax
from jax.experimental import pallas as pl
from jax.experimental.pallas import tpu as pltpu
from jax.experimental.pallas import tpu_sc as plsc

_BATCH = 16384
_HIST = 200
_EMBED = 32
_B = _BATCH * _HIST
_LANES = 128
_SUB = 8
_CHUNK = _SUB * _LANES
_NW = 32
_NCHUNKS = _B // _CHUNK
_CPW = _NCHUNKS // _NW
_HG = _HIST // _SUB
_EG = _EMBED // 8


def _body(idx_hbm, table_hbm, out_hbm, idx_v, rows_v, tp_v,
          sem_i0, sem_i1, sem_g0, sem_g1, sem_t0, sem_t1):
    nc = plsc.get_sparse_core_info().num_cores
    wid = lax.axis_index("s") * nc + lax.axis_index("c")
    sem_i = (sem_i0, sem_i1)
    sem_g = (sem_g0, sem_g1)
    sem_t = (sem_t0, sem_t1)
    iota = lax.iota(jnp.int32, 16)
    half_addr = [(iota + h * 16) * _LANES for h in range(2)]

    def chunk_id(ch):
        return wid * _CPW + ch

    def start_idx(ch, slot):
        pltpu.async_copy(
            idx_hbm.at[pl.ds(chunk_id(ch) * _SUB, _SUB)], idx_v.at[slot],
            sem_i[slot])

    def wait_idx(slot):
        pltpu.make_async_copy(
            idx_hbm.at[pl.ds(0, _SUB)], idx_v.at[slot], sem_i[slot]).wait()

    def fire(slot):
        for j in range(_SUB):
            pltpu.async_copy(
                table_hbm.at[idx_v.at[slot, j]],
                rows_v.at[slot, pl.ds(j * _LANES, _LANES)],
                sem_g[slot])

    def drain_gathers(slot):
        pltpu.make_async_copy(
            table_hbm.at[pl.ds(0, _CHUNK)], rows_v.at[slot],
            sem_g[slot]).wait()

    def wait_tp(p):
        for _eg in range(_EG):
            pltpu.make_async_copy(
                tp_v.at[p, pl.ds(0, _SUB * _LANES)], out_hbm.at[0, 0, 0],
                sem_t[p]).wait()

    def transpose_store(ch, slot, g):
        cid = chunk_id(ch)
        hg = cid // _LANES
        bt = cid - hg * _LANES
        for h8 in range(_SUB):
            p = h8 % 2
            if h8 >= 2:
                wait_tp(p)
            else:
                @pl.when(g >= 1)
                def _():
                    wait_tp(p)

            def tbody(i, carry):
                for db in range(4):
                    b = i * 4 + db
                    bs = jnp.full((16,), 0, jnp.int32) + b
                    for hf in range(2):
                        v = rows_v[slot, h8 * _LANES + b,
                                   pl.ds(hf * 16, 16)]
                        plsc.store_scatter(
                            tp_v.at[p], [half_addr[hf] + bs], v)
                return carry

            lax.fori_loop(0, _LANES // 4, tbody, 0)
            h = hg * _SUB + h8
            for eg in range(_EG):
                pltpu.async_copy(
                    tp_v.at[p, pl.ds(eg * _SUB * _LANES, _SUB * _LANES)],
                    out_hbm.at[h, eg, bt], sem_t[p])

    def step(g, slot):
        @pl.when(g + 1 < _CPW)
        def _():
            wait_idx(slot ^ 1)
            fire(slot ^ 1)

        drain_gathers(slot)

        @pl.when(g + 2 < _CPW)
        def _():
            start_idx(g + 2, slot)

        transpose_store(g, slot, g)

    start_idx(0, 0)
    start_idx(1, 1)
    wait_idx(0)
    fire(0)

    def loop_body(i, carry):
        step(2 * i, 0)
        step(2 * i + 1, 1)
        return carry

    lax.fori_loop(0, _CPW // 2, loop_body, 0)
    wait_tp(0)
    wait_tp(1)


@functools.partial(jax.jit, static_argnames=())
def kernel(action, table):
    act_view = (action.astype(jnp.int32)
                .reshape(_LANES, _LANES, _HG, _SUB)
                .transpose(2, 0, 3, 1)
                .reshape(_B // _LANES, _LANES))
    mesh = plsc.VectorSubcoreMesh(core_axis_name="c", subcore_axis_name="s")
    out4 = pl.kernel(
        _body,
        out_type=jax.ShapeDtypeStruct((_HIST, _EG, _LANES, _SUB * _LANES),
                                      jnp.float32),
        mesh=mesh,
        scratch_types=[
            pltpu.VMEM((2, _SUB, _LANES), jnp.int32),
            pltpu.VMEM((2, _CHUNK, _EMBED), jnp.float32),
            pltpu.VMEM((2, _EMBED * _LANES), jnp.float32),
            pltpu.SemaphoreType.DMA,
            pltpu.SemaphoreType.DMA,
            pltpu.SemaphoreType.DMA,
            pltpu.SemaphoreType.DMA,
            pltpu.SemaphoreType.DMA,
            pltpu.SemaphoreType.DMA,
        ],
        compiler_params=pltpu.CompilerParams(use_tc_tiling_on_sc=False,
                                             needs_layout_passes=False),
    )(act_view, table)
    return (out4.reshape(_HIST, _EG, _LANES, _SUB, _LANES)
            .transpose(2, 4, 0, 1, 3)
            .reshape(_BATCH, _HIST, _EMBED))


# speedup vs baseline: 2.3131x; 1.9447x over previous
import functools

import jax
import jax.numpy as jnp
from jax import lax
from jax.experimental import pallas as pl
from jax.experimental.pallas import tpu as pltpu
from jax.experimental.pallas import tpu_sc as plsc

_BATCH = 16384
_HIST = 200
_EMBED = 32
_B = _BATCH * _HIST
_LANES = 128
_SUB = 8
_CHUNK = _SUB * _LANES
_NW = 32
_NCHUNKS = _B // _CHUNK
_CPW = _NCHUNKS // _NW
_HG = _HIST // _SUB
_EG = _EMBED // 8


def _body(idx_hbm, table_hbm, out_hbm, idx_v, rows_v, tp_v,
          sem_i0, sem_i1, sem_g0, sem_g1, sem_t0, sem_t1):
    nc = plsc.get_sparse_core_info().num_cores
    wid = lax.axis_index("s") * nc + lax.axis_index("c")
    sem_i = (sem_i0, sem_i1)
    sem_g = (sem_g0, sem_g1)
    sem_t = (sem_t0, sem_t1)
    iota = lax.iota(jnp.int32, 16)
    half_e = [iota + h * 16 for h in range(2)]

    def chunk_id(ch):
        return wid * _CPW + ch

    def start_idx(ch, slot):
        pltpu.async_copy(
            idx_hbm.at[pl.ds(chunk_id(ch) * _SUB, _SUB)], idx_v.at[slot],
            sem_i[slot])

    def wait_idx(slot):
        pltpu.make_async_copy(
            idx_hbm.at[pl.ds(0, _SUB)], idx_v.at[slot], sem_i[slot]).wait()

    def fire(slot):
        for j in range(_SUB):
            pltpu.async_copy(
                table_hbm.at[idx_v.at[slot, j]],
                rows_v.at[slot, pl.ds(j * _LANES, _LANES)],
                sem_g[slot])

    def drain_gathers(slot):
        pltpu.make_async_copy(
            table_hbm.at[pl.ds(0, _CHUNK)], rows_v.at[slot],
            sem_g[slot]).wait()

    def wait_tp(p):
        for _eg in range(_EG):
            pltpu.make_async_copy(
                tp_v.at[p, pl.ds(0, _SUB), pl.ds(0, _LANES)],
                out_hbm.at[0, 0, 0], sem_t[p]).wait()

    def transpose_store(ch, slot, g):
        cid = chunk_id(ch)
        hg = cid // _LANES
        bt = cid - hg * _LANES
        for h8 in range(_SUB):
            p = h8 % 2
            if h8 >= 2:
                wait_tp(p)
            else:
                @pl.when(g >= 1)
                def _():
                    wait_tp(p)

            def tbody(i, carry):
                for db in range(4):
                    b = i * 4 + db
                    bs = jnp.full((16,), 0, jnp.int32) + b
                    for hf in range(2):
                        v = rows_v[slot, h8 * _LANES + b,
                                   pl.ds(hf * 16, 16)]
                        plsc.store_scatter(
                            tp_v.at[p], [half_e[hf], bs], v)
                return carry

            lax.fori_loop(0, _LANES // 4, tbody, 0)
            h = hg * _SUB + h8
            for eg in range(_EG):
                pltpu.async_copy(
                    tp_v.at[p, pl.ds(eg * _SUB, _SUB), pl.ds(0, _LANES)],
                    out_hbm.at[h, eg, bt], sem_t[p])

    def step(g, slot):
        @pl.when(g + 1 < _CPW)
        def _():
            wait_idx(slot ^ 1)
            fire(slot ^ 1)

        drain_gathers(slot)

        @pl.when(g + 2 < _CPW)
        def _():
            start_idx(g + 2, slot)

        transpose_store(g, slot, g)

    start_idx(0, 0)
    start_idx(1, 1)
    wait_idx(0)
    fire(0)

    def loop_body(i, carry):
        step(2 * i, 0)
        step(2 * i + 1, 1)
        return carry

    lax.fori_loop(0, _CPW // 2, loop_body, 0)
    wait_tp(0)
    wait_tp(1)


@functools.partial(jax.jit, static_argnames=())
def kernel(action, table):
    act_view = (action.astype(jnp.int32)
                .reshape(_LANES, _LANES, _HG, _SUB)
                .transpose(2, 0, 3, 1)
                .reshape(_B // _LANES, _LANES))
    mesh = plsc.VectorSubcoreMesh(core_axis_name="c", subcore_axis_name="s")
    out4 = pl.kernel(
        _body,
        out_type=jax.ShapeDtypeStruct((_HIST, _EG, _LANES, _SUB, _LANES),
                                      jnp.float32),
        mesh=mesh,
        scratch_types=[
            pltpu.VMEM((2, _SUB, _LANES), jnp.int32),
            pltpu.VMEM((2, _CHUNK, _EMBED), jnp.float32),
            pltpu.VMEM((2, _EMBED, _LANES + 1), jnp.float32),
            pltpu.SemaphoreType.DMA,
            pltpu.SemaphoreType.DMA,
            pltpu.SemaphoreType.DMA,
            pltpu.SemaphoreType.DMA,
            pltpu.SemaphoreType.DMA,
            pltpu.SemaphoreType.DMA,
        ],
        compiler_params=pltpu.CompilerParams(use_tc_tiling_on_sc=False,
                                             needs_layout_passes=False),
    )(act_view, table)
    return (out4.transpose(2, 4, 0, 1, 3)
            .reshape(_BATCH, _HIST, _EMBED))


# speedup vs baseline: 2.3247x; 1.0050x over previous
import functools

import jax
import jax.numpy as jnp
from jax import lax
from jax.experimental import pallas as pl
from jax.experimental.pallas import tpu as pltpu
from jax.experimental.pallas import tpu_sc as plsc

_BATCH = 16384
_HIST = 200
_EMBED = 32
_B = _BATCH * _HIST
_LANES = 128
_SUB = 8
_CHUNK = _SUB * _LANES
_NW = 32
_NCHUNKS = _B // _CHUNK
_CPW = _NCHUNKS // _NW
_HG = _HIST // _SUB
_EG = _EMBED // 8


def _body(idx_hbm, table_hbm, out_hbm, idx_v, rows_v, tp_v,
          sem_i0, sem_i1, sem_g0, sem_g1, sem_t):
    nc = plsc.get_sparse_core_info().num_cores
    wid = lax.axis_index("s") * nc + lax.axis_index("c")
    sem_i = (sem_i0, sem_i1)
    sem_g = (sem_g0, sem_g1)
    iota = lax.iota(jnp.int32, 16)
    half_e = [iota + h * 16 for h in range(2)]

    def chunk_id(ch):
        return wid * _CPW + ch

    def start_idx(ch, slot):
        pltpu.async_copy(
            idx_hbm.at[pl.ds(chunk_id(ch) * _SUB, _SUB)], idx_v.at[slot],
            sem_i[slot])

    def wait_idx(slot):
        pltpu.make_async_copy(
            idx_hbm.at[pl.ds(0, _SUB)], idx_v.at[slot], sem_i[slot]).wait()

    def fire(slot):
        for j in range(_SUB):
            pltpu.async_copy(
                table_hbm.at[idx_v.at[slot, j]],
                rows_v.at[slot, pl.ds(j * _LANES, _LANES)],
                sem_g[slot])

    def drain_gathers(slot):
        pltpu.make_async_copy(
            table_hbm.at[pl.ds(0, _CHUNK)], rows_v.at[slot],
            sem_g[slot]).wait()

    def drain_tp():
        for _ in range(_SUB * _EG):
            pltpu.make_async_copy(
                tp_v.at[0, pl.ds(0, _SUB), pl.ds(0, _LANES)],
                out_hbm.at[0, 0, 0], sem_t).wait()

    def transpose_store(ch, slot, g):
        cid = chunk_id(ch)
        hg = cid // _LANES
        bt = cid - hg * _LANES

        @pl.when(g >= 1)
        def _():
            drain_tp()

        for h8 in range(_SUB):

            def tbody(i, carry):
                for db in range(8):
                    b = i * 8 + db
                    bs = jnp.full((16,), 0, jnp.int32) + b
                    for hf in range(2):
                        v = rows_v[slot, h8 * _LANES + b,
                                   pl.ds(hf * 16, 16)]
                        plsc.store_scatter(
                            tp_v.at[h8], [half_e[hf], bs], v)
                return carry

            lax.fori_loop(0, _LANES // 8, tbody, 0)
            h = hg * _SUB + h8
            for eg in range(_EG):
                pltpu.async_copy(
                    tp_v.at[h8, pl.ds(eg * _SUB, _SUB), pl.ds(0, _LANES)],
                    out_hbm.at[h, eg, bt], sem_t)

    def step(g, slot):
        @pl.when(g + 1 < _CPW)
        def _():
            wait_idx(slot ^ 1)
            fire(slot ^ 1)

        drain_gathers(slot)

        @pl.when(g + 2 < _CPW)
        def _():
            start_idx(g + 2, slot)

        transpose_store(g, slot, g)

    start_idx(0, 0)
    start_idx(1, 1)
    wait_idx(0)
    fire(0)

    def loop_body(i, carry):
        step(2 * i, 0)
        step(2 * i + 1, 1)
        return carry

    lax.fori_loop(0, _CPW // 2, loop_body, 0)
    drain_tp()


@functools.partial(jax.jit, static_argnames=())
def kernel(action, table):
    act_view = (action.astype(jnp.int32)
                .reshape(_LANES, _LANES, _HG, _SUB)
                .transpose(2, 0, 3, 1)
                .reshape(_B // _LANES, _LANES))
    mesh = plsc.VectorSubcoreMesh(core_axis_name="c", subcore_axis_name="s")
    out4 = pl.kernel(
        _body,
        out_type=jax.ShapeDtypeStruct((_HIST, _EG, _LANES, _SUB, _LANES),
                                      jnp.float32),
        mesh=mesh,
        scratch_types=[
            pltpu.VMEM((2, _SUB, _LANES), jnp.int32),
            pltpu.VMEM((2, _CHUNK, _EMBED), jnp.float32),
            pltpu.VMEM((_SUB, _EMBED, _LANES + 1), jnp.float32),
            pltpu.SemaphoreType.DMA,
            pltpu.SemaphoreType.DMA,
            pltpu.SemaphoreType.DMA,
            pltpu.SemaphoreType.DMA,
            pltpu.SemaphoreType.DMA,
        ],
        compiler_params=pltpu.CompilerParams(use_tc_tiling_on_sc=False,
                                             needs_layout_passes=False),
    )(act_view, table)
    return (out4.transpose(2, 4, 0, 1, 3)
            .reshape(_BATCH, _HIST, _EMBED))


# speedup vs baseline: 3.7369x; 1.6075x over previous
import functools

import jax
import jax.numpy as jnp
from jax import lax
from jax.experimental import pallas as pl
from jax.experimental.pallas import tpu as pltpu
from jax.experimental.pallas import tpu_sc as plsc

_BATCH = 16384
_HIST = 200
_EMBED = 32
_B = _BATCH * _HIST
_LANES = 128
_SUB = 8
_CHUNK = _SUB * _LANES
_NW = 32
_NCHUNKS = _B // _CHUNK
_CPW = _NCHUNKS // _NW
_HG = _HIST // _SUB
_EG = _EMBED // 8


def _body(idx_hbm, table_hbm, out_hbm, idx_v, rows_v, tp_v,
          sem_i0, sem_i1, sem_g0, sem_g1, sem_t):
    nc = plsc.get_sparse_core_info().num_cores
    wid = lax.axis_index("s") * nc + lax.axis_index("c")
    sem_i = (sem_i0, sem_i1)
    sem_g = (sem_g0, sem_g1)
    iota = lax.iota(jnp.int32, 16)
    half_e = [iota + h * 16 for h in range(2)]

    def chunk_id(ch):
        return wid * _CPW + ch

    def start_idx(ch, slot):
        pltpu.async_copy(
            idx_hbm.at[pl.ds(chunk_id(ch) * _SUB, _SUB)], idx_v.at[slot],
            sem_i[slot])

    def wait_idx(slot):
        pltpu.make_async_copy(
            idx_hbm.at[pl.ds(0, _SUB)], idx_v.at[slot], sem_i[slot]).wait()

    def fire(slot):
        for j in range(_SUB):
            pltpu.async_copy(
                table_hbm.at[idx_v.at[slot, j]],
                rows_v.at[slot, pl.ds(j * _LANES, _LANES)],
                sem_g[slot])

    def drain_gathers(slot):
        pltpu.make_async_copy(
            table_hbm.at[pl.ds(0, _CHUNK)], rows_v.at[slot],
            sem_g[slot]).wait()

    def drain_tp():
        for _ in range(_SUB * _EG):
            pltpu.make_async_copy(
                tp_v.at[0, pl.ds(0, _SUB), pl.ds(0, _LANES)],
                out_hbm.at[0, 0, 0], sem_t).wait()

    def transpose_store(ch, slot, g):
        cid = chunk_id(ch)
        hg = cid // _LANES
        bt = cid - hg * _LANES

        @pl.when(g >= 1)
        def _():
            drain_tp()

        for h8 in range(_SUB):

            @plsc.parallel_loop(0, _LANES, 1, unroll=8)
            def _(b):
                bs = jnp.full((16,), 0, jnp.int32) + b
                for hf in range(2):
                    v = rows_v[slot, h8 * _LANES + b, pl.ds(hf * 16, 16)]
                    plsc.store_scatter(tp_v.at[h8], [half_e[hf], bs], v)
            h = hg * _SUB + h8
            for eg in range(_EG):
                pltpu.async_copy(
                    tp_v.at[h8, pl.ds(eg * _SUB, _SUB), pl.ds(0, _LANES)],
                    out_hbm.at[h, eg, bt], sem_t)

    def step(g, slot):
        @pl.when(g + 1 < _CPW)
        def _():
            wait_idx(slot ^ 1)
            fire(slot ^ 1)

        drain_gathers(slot)

        @pl.when(g + 2 < _CPW)
        def _():
            start_idx(g + 2, slot)

        transpose_store(g, slot, g)

    start_idx(0, 0)
    start_idx(1, 1)
    wait_idx(0)
    fire(0)

    def loop_body(i, carry):
        step(2 * i, 0)
        step(2 * i + 1, 1)
        return carry

    lax.fori_loop(0, _CPW // 2, loop_body, 0)
    drain_tp()


@functools.partial(jax.jit, static_argnames=())
def kernel(action, table):
    act_view = (action.astype(jnp.int32)
                .reshape(_LANES, _LANES, _HG, _SUB)
                .transpose(2, 0, 3, 1)
                .reshape(_B // _LANES, _LANES))
    mesh = plsc.VectorSubcoreMesh(core_axis_name="c", subcore_axis_name="s")
    out4 = pl.kernel(
        _body,
        out_type=jax.ShapeDtypeStruct((_HIST, _EG, _LANES, _SUB, _LANES),
                                      jnp.float32),
        mesh=mesh,
        scratch_types=[
            pltpu.VMEM((2, _SUB, _LANES), jnp.int32),
            pltpu.VMEM((2, _CHUNK, _EMBED), jnp.float32),
            pltpu.VMEM((_SUB, _EMBED, _LANES + 1), jnp.float32),
            pltpu.SemaphoreType.DMA,
            pltpu.SemaphoreType.DMA,
            pltpu.SemaphoreType.DMA,
            pltpu.SemaphoreType.DMA,
            pltpu.SemaphoreType.DMA,
        ],
        compiler_params=pltpu.CompilerParams(use_tc_tiling_on_sc=False,
                                             needs_layout_passes=False),
    )(act_view, table)
    return (out4.transpose(2, 4, 0, 1, 3)
            .reshape(_BATCH, _HIST, _EMBED))
